# baseline (speedup 1.0000x reference)
import numpy as np
import jax
import jax.numpy as jnp
from jax import lax
from jax.experimental import pallas as pl
from jax.experimental.pallas import tpu as pltpu
from jax.experimental.pallas import tpu_sc as plsc

N = 10000
M = 2500
D = 128
B = 128
E = 320000
EEE = 80000

NC = 2
NS = 16
NW = NC * NS
CHUNK = 128

E_PAD = 327680
EE_PAD = 98304

SD_N = 10112
SD_M = 2560
SD16_N = 10240
SD16_M = 3072
OR_N = SD16_N * 16 // 128
OR_M = SD16_M * 16 // 128

F32 = jnp.float32


def _mesh():
    return plsc.VectorSubcoreMesh(core_axis_name="c", subcore_axis_name="s",
                                  num_cores=NC, num_subcores=NS)


def _pad_pair(src, dst, epad, table_rows, seg_rows):
    p = epad - src.shape[0]
    ar = jnp.arange(p, dtype=jnp.int32)
    src_p = jnp.concatenate([src.astype(jnp.int32), ar % min(2048, table_rows)])
    dst_p = jnp.concatenate([dst.astype(jnp.int32), seg_rows + (ar % 8)])
    return src_p, dst_p.reshape(epad // CHUNK, CHUNK)


def _make_segsum(ops):
    n_ops = len(ops)
    n_tab = sum(1 for o in ops if o[0])
    tab_pos = {}
    for i, o in enumerate(ops):
        if o[0]:
            tab_pos[i] = len(tab_pos)

    def body(*refs):
        tables = refs[0:n_tab]
        srcs = refs[n_tab:2 * n_tab]
        dsts = refs[2 * n_tab:2 * n_tab + n_ops]
        zeros = refs[2 * n_tab + n_ops]
        outs = refs[2 * n_tab + n_ops + 1: 2 * n_tab + 2 * n_ops + 1]
        accum = refs[2 * n_tab + 2 * n_ops + 1]
        (sidx, didx, rows0, rows1,
         sem0, sem1, sem2, sem3) = refs[2 * n_tab + 2 * n_ops + 2:]
        c = lax.axis_index("c")
        s = lax.axis_index("s")
        wid = s * NC + c

        if n_tab < n_ops:
            ones16 = jnp.ones((16,), F32)

            def fill(r):
                for jj in range(8):
                    rows0[r, pl.ds(16 * jj, 16)] = ones16
            pl.loop(0, CHUNK)(fill)

        for k in range(n_ops):
            has_tab, sd, epad, grp = ops[k]
            rpt = sd // NS
            pltpu.sync_copy(zeros.at[pl.ds(0, rpt)],
                            accum.at[pl.ds(s * rpt, rpt)])
            plsc.subcore_barrier()

            cw = epad // NW // CHUNK
            ng = cw // grp
            base = wid * cw

            if has_tab:
                t = tab_pos[k]

                def group_body(g, t=t, k=k, base=base, grp=grp):
                    gb = base + g * grp
                    pltpu.sync_copy(srcs[t].at[pl.ds(gb * CHUNK, grp * CHUNK)],
                                    sidx.at[pl.ds(0, grp * CHUNK)])
                    pltpu.sync_copy(dsts[k].at[pl.ds(gb, grp)],
                                    didx.at[pl.ds(0, grp)])
                    rbufs = (rows0, rows1)
                    gsems = (sem0, sem1)
                    ssems = (sem2, sem3)
                    gd = [None] * grp
                    sd = [None] * grp
                    gd[0] = pltpu.async_copy(
                        tables[t].at[sidx.at[pl.ds(0, CHUNK)]], rbufs[0],
                        gsems[0])
                    for j in range(grp):
                        gd[j].wait()
                        sd[j] = pltpu.async_copy(
                            rbufs[j % 2], accum.at[didx.at[j]], ssems[j % 2],
                            add=True)
                        if j + 1 < grp:
                            if j >= 1:
                                sd[j - 1].wait()
                            gd[j + 1] = pltpu.async_copy(
                                tables[t].at[
                                    sidx.at[pl.ds((j + 1) * CHUNK, CHUNK)]],
                                rbufs[(j + 1) % 2], gsems[(j + 1) % 2])
                    sd[grp - 2].wait()
                    sd[grp - 1].wait()
            else:
                def group_body(g, k=k, base=base, grp=grp):
                    gb = base + g * grp
                    pltpu.sync_copy(dsts[k].at[pl.ds(gb, grp)],
                                    didx.at[pl.ds(0, grp)])
                    for j in range(grp):
                        pltpu.sync_copy(rows0, accum.at[didx.at[j]],
                                        add=True)

            pl.loop(0, ng)(group_body)
            plsc.subcore_barrier()

            pltpu.sync_copy(accum.at[pl.ds(s * rpt, rpt)],
                            outs[k].at[c].at[pl.ds(s * rpt, rpt)])
            plsc.subcore_barrier()

    max_grp = max(o[3] for o in ops)
    max_sd = max(o[1] for o in ops)
    scratch = (
        [pltpu.VMEM_SHARED((max_sd, D), F32)] +
        [pltpu.VMEM((max_grp * CHUNK,), jnp.int32),
         pltpu.VMEM((max_grp, CHUNK), jnp.int32),
         pltpu.VMEM((CHUNK, D), F32),
         pltpu.VMEM((CHUNK, D), F32),
         pltpu.SemaphoreType.DMA,
         pltpu.SemaphoreType.DMA,
         pltpu.SemaphoreType.DMA,
         pltpu.SemaphoreType.DMA])

    out_type = [jax.ShapeDtypeStruct((NC, o[1], D), F32) for o in ops]
    return pl.kernel(body, out_type=out_type, mesh=_mesh(),
                     scratch_types=scratch)


def _segsum(op_args):
    ops = [(a[0] is not None, a[3], a[2].shape[0] * CHUNK, a[4])
           for a in op_args]
    assert all(o[0] for o in ops) or not any(
        o[0] for o in ops[:max(i for i, o in enumerate(ops) if not o[0]) + 1]
    ), "count ops must precede table ops"
    fn = _make_segsum(ops)
    zeros = jnp.zeros((max(o[1] for o in ops) // NS, D), F32)
    args = ([a[0] for a in op_args if a[0] is not None]
            + [a[1] for a in op_args if a[1] is not None]
            + [a[2] for a in op_args] + [zeros])
    res = fn(*args)
    return res if isinstance(res, (list, tuple)) else (res,)


SQRT2 = float(np.sqrt(2.0))


def _gelu(z):
    return 0.5 * z * (1.0 + lax.erf(z / SQRT2))


def _layer_norm(z, g, b):
    mu = jnp.mean(z, axis=-1, keepdims=True)
    v = jnp.mean((z - mu) ** 2, axis=-1, keepdims=True)
    return (z - mu) / jnp.sqrt(v + 1e-5) * g + b


def _bt_loss(z1, z2, n):
    m1 = jnp.mean(z1, axis=0, keepdims=True)
    s1 = jnp.sqrt(jnp.sum((z1 - m1) ** 2, axis=0, keepdims=True) / (n - 1))
    z1 = (z1 - m1) / s1
    m2 = jnp.mean(z2, axis=0, keepdims=True)
    s2 = jnp.sqrt(jnp.sum((z2 - m2) ** 2, axis=0, keepdims=True) / (n - 1))
    z2 = (z2 - m2) / s2

    def bn(z):
        mu = jnp.mean(z, axis=0, keepdims=True)
        v = jnp.mean((z - mu) ** 2, axis=0, keepdims=True)
        return (z - mu) / jnp.sqrt(v + 1e-5)

    cc = lax.dot_general(bn(z1), bn(z2), (((0,), (0,)), ((), ())),
                         preferred_element_type=F32) / float(B)
    eye = (lax.broadcasted_iota(jnp.int32, (D, D), 0)
           == lax.broadcasted_iota(jnp.int32, (D, D), 1)).astype(F32)
    on = jnp.sum(((cc - 1.0) * eye) ** 2)
    csq = cc * cc
    off = jnp.sum(csq) - jnp.sum(csq * eye)
    return on + 0.005 * off


def _cnt(ref):
    return jnp.maximum(ref[0, :, 0:1] + ref[1, :, 0:1], 1.0)


def _t1p(a1p, cep, dxp, dep, x, hf, w):
    def body(a_r, c_r, dx_r, de_r, x_r, hf_r, w_r, xs_r, hs_r, eagg_r):
        xs_r[...] = x_r[...] * lax.rsqrt(_cnt(dx_r))
        hs_r[...] = hf_r[...] * lax.rsqrt(_cnt(de_r))
        sums = a_r[0, :M, :] + a_r[1, :M, :]
        eagg_r[...] = jnp.dot(sums / _cnt(c_r), w_r[...],
                              preferred_element_type=F32)
    return pl.pallas_call(body, out_shape=[
        jax.ShapeDtypeStruct((N, D), F32),
        jax.ShapeDtypeStruct((M, D), F32),
        jax.ShapeDtypeStruct((M, D), F32)])(a1p, cep, dxp, dep, x, hf, w)


def _t1(a1p, cep, w):
    def body(a_r, c_r, w_r, o_r):
        sums = a_r[0, :M, :] + a_r[1, :M, :]
        o_r[...] = jnp.dot(sums / _cnt(c_r), w_r[...],
                           preferred_element_type=F32)
    return pl.pallas_call(body, out_shape=jax.ShapeDtypeStruct((M, D), F32))(
        a1p, cep, w)


def _comb(p, v, s_rows, mode):
    sd = p.shape[1]
    rb = 1264 if sd == SD_N else 1280
    def body(p_r, v_r, o_r):
        cnt = jnp.maximum(v_r[0, :, 0:1] + v_r[1, :, 0:1], 1.0)
        sums = p_r[0] + p_r[1]
        o_r[...] = sums / cnt if mode == "mean" else sums * lax.rsqrt(cnt)
    out = pl.pallas_call(
        body,
        grid=(sd // rb,),
        in_specs=[pl.BlockSpec((NC, rb, D), lambda i: (0, i, 0)),
                  pl.BlockSpec((NC, rb, D), lambda i: (0, i, 0))],
        out_specs=pl.BlockSpec((rb, D), lambda i: (i, 0)),
        out_shape=jax.ShapeDtypeStruct((sd, D), F32))(p, v)
    return out[:s_rows]


def _t2z(a2c, a3c, x_res, dxp, we2n, wx, bx, lng, lnb, want_next):
    def body(a2_r, a3_r, x_r, dx_r, we2n_r, wx_r, bx_r, lng_r, lnb_r, *outs):
        z = jnp.dot(a2_r[...], we2n_r[...], preferred_element_type=F32)
        z_imp = jnp.dot(a3_r[...], wx_r[...],
                        preferred_element_type=F32) + bx_r[...]
        bt = _bt_loss(z_imp, z, N)
        zo = _gelu(_layer_norm(z, lng_r[...], lnb_r[...])) + x_r[...]
        outs[0][...] = zo
        outs[-1][...] = bt.reshape(1, 1)
        if want_next:
            outs[1][...] = zo * lax.rsqrt(_cnt(dx_r))
    shapes = [jax.ShapeDtypeStruct((N, D), F32)]
    if want_next:
        shapes.append(jax.ShapeDtypeStruct((N, D), F32))
    shapes.append(jax.ShapeDtypeStruct((1, 1), F32))
    return pl.pallas_call(body, out_shape=shapes)(
        a2c, a3c, x_res, dxp, we2n, wx, bx, lng, lnb)


def _t2e(a4c, eagg, hf_res, dep, we, be, lng, lnb, want_next):
    def body(a4_r, eagg_r, hf_r, de_r, we_r, be_r, lng_r, lnb_r, *outs):
        e_imp = jnp.dot(a4_r[...], we_r[...],
                        preferred_element_type=F32) + be_r[...]
        e = eagg_r[...]
        bt = _bt_loss(e_imp, e, M)
        eo = _gelu(_layer_norm(e, lng_r[...], lnb_r[...])) + hf_r[...]
        outs[0][...] = eo
        outs[-1][...] = bt.reshape(1, 1)
        if want_next:
            outs[1][...] = eo * lax.rsqrt(_cnt(de_r))
    shapes = [jax.ShapeDtypeStruct((M, D), F32)]
    if want_next:
        shapes.append(jax.ShapeDtypeStruct((M, D), F32))
    shapes.append(jax.ShapeDtypeStruct((1, 1), F32))
    return pl.pallas_call(body, out_shape=shapes)(
        a4c, eagg, hf_res, dep, we, be, lng, lnb)


def _t3(zo, eo, nb, eb, wfus, bfus, wd1, bd1, wd2, bd2,
        btz0, bte0, btz1, bte1):
    def body(zo_r, eo_r, nb_r, eb_r, wfus_r, bfus_r, wd1_r, bd1_r, wd2_r,
             bd2_r, bz0_r, be0_r, bz1_r, be1_r, out_r, btt_r):
        def pool(idx, data):
            oh = (idx.reshape(idx.shape[0], 1)
                  == lax.broadcasted_iota(jnp.int32, (1, B), 1)).astype(F32)
            sums = lax.dot_general(oh, data, (((0,), (0,)), ((), ())),
                                   preferred_element_type=F32)
            cnt = jnp.maximum(jnp.sum(oh, axis=0), 1.0).reshape(B, 1)
            return sums / cnt

        zg = pool(nb_r[...], zo_r[...])
        eg = pool(eb_r[...], eo_r[...])
        graph = jnp.dot(jnp.concatenate([zg, eg], axis=1), wfus_r[...],
                        preferred_element_type=F32) + bfus_r[...]
        h = jnp.dot(graph, wd1_r[...], preferred_element_type=F32) + bd1_r[...]
        out_r[...] = jnp.dot(h, wd2_r[...],
                             preferred_element_type=F32) + bd2_r[...]
        btt_r[...] = ((bz0_r[...] + be0_r[...]) + bz1_r[...]) + be1_r[...]
    return pl.pallas_call(body, out_shape=[
        jax.ShapeDtypeStruct((B, 2), F32),
        jax.ShapeDtypeStruct((1, 1), F32)])(
        zo, eo, nb, eb, wfus, bfus, wd1, bd1, wd2, bd2,
        btz0, bte0, btz1, bte1)


def kernel(x, hyperedge_feature, node_coord, W_n2e, W_e2n, w_coord, W_x, b_x,
           W_e, b_e, ln_g, ln_b, W_fus, b_fus, W_dec1, b_dec1, W_dec2, b_dec2,
           hyperedge_node_idx, hyperedge_edge_idx, node_batch_idx,
           hyperedge_batch_idx, XX_index, EE_index):
    hf = hyperedge_feature
    n_src, e_dst2 = _pad_pair(hyperedge_node_idx, hyperedge_edge_idx,
                              E_PAD, N, M)
    e_src, n_dst2 = _pad_pair(hyperedge_edge_idx, hyperedge_node_idx,
                              E_PAD, M, N)
    xx_src, xx_dst2 = _pad_pair(XX_index[0], XX_index[1], E_PAD, N, N)
    ee_src, ee_dst2 = _pad_pair(EE_index[0], EE_index[1], EE_PAD, M, M)

    ce_f, cn_f, dx_f, de_f, a1p = _segsum([
        (None, None, e_dst2, SD_M, 8),
        (None, None, n_dst2, SD_N, 8),
        (None, None, xx_dst2, SD_N, 8),
        (None, None, ee_dst2, SD_M, 8),
        (x, n_src, e_dst2, SD_M, 8),
    ])
    cep = ce_f[:, :M, 0:1]
    dxp = dx_f[:, :N, 0:1]
    dep = de_f[:, :M, 0:1]

    xs0, hs0, eagg = _t1p(a1p, cep, dxp, dep, x, hf, W_n2e[0])

    a3p, a4p, a2p = _segsum([
        (xs0, xx_src, xx_dst2, SD_N, 8),
        (hs0, ee_src, ee_dst2, SD_M, 8),
        (eagg, e_src, n_dst2, SD_N, 8),
    ])
    a2c = _comb(a2p, cn_f, N, "mean")
    a3c = _comb(a3p, dx_f, N, "rsqrt")
    a4c = _comb(a4p, de_f, M, "rsqrt")
    zo0, xs1, btz0 = _t2z(a2c, a3c, x, dxp, W_e2n[0], W_x[0], b_x[0],
                          ln_g[0], ln_b[0], True)
    eo0, hs1, bte0 = _t2e(a4c, eagg, hf, dep, W_e[0], b_e[0],
                          ln_g[0], ln_b[0], True)

    a1p1, a3p1, a4p1 = _segsum([
        (zo0, n_src, e_dst2, SD_M, 8),
        (xs1, xx_src, xx_dst2, SD_N, 8),
        (hs1, ee_src, ee_dst2, SD_M, 8),
    ])
    eagg1 = _t1(a1p1, cep, W_n2e[1])

    (a2p1,) = _segsum([(eagg1, e_src, n_dst2, SD_N, 8)])
    a2c1 = _comb(a2p1, cn_f, N, "mean")
    a3c1 = _comb(a3p1, dx_f, N, "rsqrt")
    a4c1 = _comb(a4p1, de_f, M, "rsqrt")
    zo1, btz1 = _t2z(a2c1, a3c1, zo0, dxp, W_e2n[1], W_x[1], b_x[1],
                     ln_g[1], ln_b[1], False)
    eo1, bte1 = _t2e(a4c1, eagg1, eo0, dep, W_e[1], b_e[1],
                     ln_g[1], ln_b[1], False)

    out, btt = _t3(zo1, eo1, node_batch_idx.astype(jnp.int32),
                   hyperedge_batch_idx.astype(jnp.int32),
                   W_fus[1], b_fus[1], W_dec1, b_dec1, W_dec2, b_dec2,
                   btz0, bte0, btz1, bte1)
    return out, btt.reshape(())


# speedup vs baseline: 1.0580x; 1.0580x over previous
import numpy as np
import jax
import jax.numpy as jnp
from jax import lax
from jax.experimental import pallas as pl
from jax.experimental.pallas import tpu as pltpu
from jax.experimental.pallas import tpu_sc as plsc

N = 10000
M = 2500
D = 128
B = 128
E = 320000
EEE = 80000

NC = 2
NS = 16
NW = NC * NS
CHUNK = 128

E_PAD = 327680
EE_PAD = 98304

SD_N = 10112
SD_M = 2560
SD16_N = 10240
SD16_M = 3072
OR_N = SD16_N * 16 // 128
OR_M = SD16_M * 16 // 128

F32 = jnp.float32


def _mesh():
    return plsc.VectorSubcoreMesh(core_axis_name="c", subcore_axis_name="s",
                                  num_cores=NC, num_subcores=NS)


def _pad_pair(src, dst, epad, table_rows, seg_rows):
    p = epad - src.shape[0]
    ar = jnp.arange(p, dtype=jnp.int32)
    src_p = jnp.concatenate([src.astype(jnp.int32), ar % min(2048, table_rows)])
    dst_p = jnp.concatenate([dst.astype(jnp.int32), seg_rows + (ar % 8)])
    return src_p, dst_p.reshape(epad // CHUNK, CHUNK)


def _make_segsum(ops):
    n_ops = len(ops)
    n_tab = sum(1 for o in ops if o[0])
    tab_pos = {}
    for i, o in enumerate(ops):
        if o[0]:
            tab_pos[i] = len(tab_pos)

    def body(*refs):
        tables = refs[0:n_tab]
        srcs = refs[n_tab:2 * n_tab]
        dsts = refs[2 * n_tab:2 * n_tab + n_ops]
        zeros = refs[2 * n_tab + n_ops]
        outs = refs[2 * n_tab + n_ops + 1: 2 * n_tab + 2 * n_ops + 1]
        it = iter(refs[2 * n_tab + 2 * n_ops + 1:])
        accum = next(it)
        sidx, didx, rows0, rows1, sem0, sem1 = (next(it), next(it), next(it),
                                                next(it), next(it), next(it))
        c = lax.axis_index("c")
        s = lax.axis_index("s")
        wid = s * NC + c

        if n_tab < n_ops:
            ones16 = jnp.ones((16,), F32)

            def fill(r):
                for jj in range(8):
                    rows0[r, pl.ds(16 * jj, 16)] = ones16
            pl.loop(0, CHUNK)(fill)

        for k in range(n_ops):
            has_tab, sd, epad, grp = ops[k]
            cw = epad // NW // CHUNK
            ng = cw // grp
            base = wid * cw

            if has_tab:
                rpt = sd // NS
                pltpu.sync_copy(zeros.at[pl.ds(0, rpt)],
                                accum.at[pl.ds(s * rpt, rpt)])
                plsc.subcore_barrier()
                t = tab_pos[k]

                def group_body(g, t=t, k=k, base=base, grp=grp):
                    gb = base + g * grp
                    pltpu.sync_copy(srcs[t].at[pl.ds(gb * CHUNK, grp * CHUNK)],
                                    sidx.at[pl.ds(0, grp * CHUNK)])
                    pltpu.sync_copy(dsts[k].at[pl.ds(gb, grp)],
                                    didx.at[pl.ds(0, grp)])
                    rbufs = (rows0, rows1)
                    gsems = (sem0, sem1)
                    gd = [None] * grp
                    gd[0] = pltpu.async_copy(
                        tables[t].at[sidx.at[pl.ds(0, CHUNK)]], rbufs[0],
                        gsems[0])
                    for j in range(grp):
                        if j + 1 < grp:
                            gd[j + 1] = pltpu.async_copy(
                                tables[t].at[
                                    sidx.at[pl.ds((j + 1) * CHUNK, CHUNK)]],
                                rbufs[(j + 1) % 2], gsems[(j + 1) % 2])
                        gd[j].wait()
                        pltpu.sync_copy(rbufs[j % 2], accum.at[didx.at[j]],
                                        add=True)

                pl.loop(0, ng)(group_body)
                plsc.subcore_barrier()

                pltpu.sync_copy(accum.at[pl.ds(s * rpt, rpt)],
                                outs[k].at[c].at[pl.ds(s * rpt, rpt)])
                plsc.subcore_barrier()
            else:
                rpt = sd // NS
                pltpu.sync_copy(zeros.at[pl.ds(0, rpt)],
                                accum.at[pl.ds(s * rpt, rpt)])
                plsc.subcore_barrier()

                def group_body(g, k=k, base=base, grp=grp):
                    gb = base + g * grp
                    pltpu.sync_copy(dsts[k].at[pl.ds(gb, grp)],
                                    didx.at[pl.ds(0, grp)])
                    for j in range(grp):
                        pltpu.sync_copy(rows0, accum.at[didx.at[j]],
                                        add=True)

                pl.loop(0, ng)(group_body)
                plsc.subcore_barrier()

                pltpu.sync_copy(accum.at[pl.ds(s * rpt, rpt)],
                                outs[k].at[c].at[pl.ds(s * rpt, rpt)])
                plsc.subcore_barrier()

    max_grp = max(o[3] for o in ops)
    max_sd = max(o[1] for o in ops)
    scratch = [pltpu.VMEM_SHARED((max_sd, D), F32),
               pltpu.VMEM((max_grp * CHUNK,), jnp.int32),
               pltpu.VMEM((max_grp, CHUNK), jnp.int32),
               pltpu.VMEM((CHUNK, D), F32),
               pltpu.VMEM((CHUNK, D), F32),
               pltpu.SemaphoreType.DMA,
               pltpu.SemaphoreType.DMA]

    out_type = [jax.ShapeDtypeStruct((NC, o[1], D), F32) for o in ops]
    return pl.kernel(body, out_type=out_type, mesh=_mesh(),
                     scratch_types=scratch)


def _segsum(op_args):
    ops = [(a[0] is not None, a[3], a[2].shape[0] * CHUNK, a[4])
           for a in op_args]
    fn = _make_segsum(ops)
    zeros = jnp.zeros((max(o[1] for o in ops) // NS, D), F32)
    args = ([a[0] for a in op_args if a[0] is not None]
            + [a[1] for a in op_args if a[1] is not None]
            + [a[2] for a in op_args] + [zeros])
    res = fn(*args)
    return res if isinstance(res, (list, tuple)) else (res,)


SQRT2 = float(np.sqrt(2.0))


def _gelu(z):
    return 0.5 * z * (1.0 + lax.erf(z / SQRT2))


def _layer_norm(z, g, b):
    mu = jnp.mean(z, axis=-1, keepdims=True)
    v = jnp.mean((z - mu) ** 2, axis=-1, keepdims=True)
    return (z - mu) / jnp.sqrt(v + 1e-5) * g + b


def _bt_loss(z1, z2, n):
    m1 = jnp.mean(z1, axis=0, keepdims=True)
    s1 = jnp.sqrt(jnp.sum((z1 - m1) ** 2, axis=0, keepdims=True) / (n - 1))
    z1 = (z1 - m1) / s1
    m2 = jnp.mean(z2, axis=0, keepdims=True)
    s2 = jnp.sqrt(jnp.sum((z2 - m2) ** 2, axis=0, keepdims=True) / (n - 1))
    z2 = (z2 - m2) / s2

    def bn(z):
        mu = jnp.mean(z, axis=0, keepdims=True)
        v = jnp.mean((z - mu) ** 2, axis=0, keepdims=True)
        return (z - mu) / jnp.sqrt(v + 1e-5)

    cc = lax.dot_general(bn(z1), bn(z2), (((0,), (0,)), ((), ())),
                         preferred_element_type=F32,
                         precision=lax.Precision.HIGHEST) / float(B)
    eye = (lax.broadcasted_iota(jnp.int32, (D, D), 0)
           == lax.broadcasted_iota(jnp.int32, (D, D), 1)).astype(F32)
    on = jnp.sum(((cc - 1.0) * eye) ** 2)
    csq = cc * cc
    off = jnp.sum(csq) - jnp.sum(csq * eye)
    return on + 0.005 * off


def _cnt(ref):
    return jnp.maximum(ref[0, :, 0:1] + ref[1, :, 0:1], 1.0)


def _t1p(a1p, cep, dxp, dep, x, hf, w):
    def body(a_r, c_r, dx_r, de_r, x_r, hf_r, w_r, xs_r, hs_r, eagg_r):
        xs_r[...] = x_r[...] * lax.rsqrt(_cnt(dx_r))
        hs_r[...] = hf_r[...] * lax.rsqrt(_cnt(de_r))
        sums = a_r[0, :M, :] + a_r[1, :M, :]
        eagg_r[...] = jnp.dot(sums / _cnt(c_r), w_r[...],
                              preferred_element_type=F32)
    return pl.pallas_call(body, out_shape=[
        jax.ShapeDtypeStruct((N, D), F32),
        jax.ShapeDtypeStruct((M, D), F32),
        jax.ShapeDtypeStruct((M, D), F32)])(a1p, cep, dxp, dep, x, hf, w)


def _t1(a1p, cep, w):
    def body(a_r, c_r, w_r, o_r):
        sums = a_r[0, :M, :] + a_r[1, :M, :]
        o_r[...] = jnp.dot(sums / _cnt(c_r), w_r[...],
                           preferred_element_type=F32)
    return pl.pallas_call(body, out_shape=jax.ShapeDtypeStruct((M, D), F32))(
        a1p, cep, w)


def _comb(p, v, s_rows, mode):
    sd = p.shape[1]
    rb = 1264 if sd == SD_N else 1280
    def body(p_r, v_r, o_r):
        cnt = jnp.maximum(v_r[0] + v_r[1], 1.0)
        sums = p_r[0] + p_r[1]
        o_r[...] = sums / cnt if mode == "mean" else sums * lax.rsqrt(cnt)
    out = pl.pallas_call(
        body,
        grid=(sd // rb,),
        in_specs=[pl.BlockSpec((NC, rb, D), lambda i: (0, i, 0)),
                  pl.BlockSpec((NC, rb, 1), lambda i: (0, i, 0))],
        out_specs=pl.BlockSpec((rb, D), lambda i: (i, 0)),
        out_shape=jax.ShapeDtypeStruct((sd, D), F32))(p, v)
    return out[:s_rows]


def _t2z(a2c, a3c, x_res, dxp, we2n, wx, bx, lng, lnb, want_next):
    def body(a2_r, a3_r, x_r, dx_r, we2n_r, wx_r, bx_r, lng_r, lnb_r, *outs):
        z = jnp.dot(a2_r[...], we2n_r[...], preferred_element_type=F32)
        z_imp = jnp.dot(a3_r[...], wx_r[...],
                        preferred_element_type=F32) + bx_r[...]
        bt = _bt_loss(z_imp, z, N)
        zo = _gelu(_layer_norm(z, lng_r[...], lnb_r[...])) + x_r[...]
        outs[0][...] = zo
        outs[-1][...] = bt.reshape(1, 1)
        if want_next:
            outs[1][...] = zo * lax.rsqrt(_cnt(dx_r))
    shapes = [jax.ShapeDtypeStruct((N, D), F32)]
    if want_next:
        shapes.append(jax.ShapeDtypeStruct((N, D), F32))
    shapes.append(jax.ShapeDtypeStruct((1, 1), F32))
    return pl.pallas_call(body, out_shape=shapes)(
        a2c, a3c, x_res, dxp, we2n, wx, bx, lng, lnb)


def _t2e(a4c, eagg, hf_res, dep, we, be, lng, lnb, want_next):
    def body(a4_r, eagg_r, hf_r, de_r, we_r, be_r, lng_r, lnb_r, *outs):
        e_imp = jnp.dot(a4_r[...], we_r[...],
                        preferred_element_type=F32) + be_r[...]
        e = eagg_r[...]
        bt = _bt_loss(e_imp, e, M)
        eo = _gelu(_layer_norm(e, lng_r[...], lnb_r[...])) + hf_r[...]
        outs[0][...] = eo
        outs[-1][...] = bt.reshape(1, 1)
        if want_next:
            outs[1][...] = eo * lax.rsqrt(_cnt(de_r))
    shapes = [jax.ShapeDtypeStruct((M, D), F32)]
    if want_next:
        shapes.append(jax.ShapeDtypeStruct((M, D), F32))
    shapes.append(jax.ShapeDtypeStruct((1, 1), F32))
    return pl.pallas_call(body, out_shape=shapes)(
        a4c, eagg, hf_res, dep, we, be, lng, lnb)


def _t3(zo, eo, nb, eb, wfus, bfus, wd1, bd1, wd2, bd2,
        btz0, bte0, btz1, bte1):
    def body(zo_r, eo_r, nb_r, eb_r, wfus_r, bfus_r, wd1_r, bd1_r, wd2_r,
             bd2_r, bz0_r, be0_r, bz1_r, be1_r, out_r, btt_r):
        def pool(idx, data):
            oh = (idx.reshape(idx.shape[0], 1)
                  == lax.broadcasted_iota(jnp.int32, (1, B), 1)).astype(F32)
            sums = lax.dot_general(oh, data, (((0,), (0,)), ((), ())),
                                   preferred_element_type=F32)
            cnt = jnp.maximum(jnp.sum(oh, axis=0), 1.0).reshape(B, 1)
            return sums / cnt

        zg = pool(nb_r[...], zo_r[...])
        eg = pool(eb_r[...], eo_r[...])
        graph = jnp.dot(jnp.concatenate([zg, eg], axis=1), wfus_r[...],
                        preferred_element_type=F32) + bfus_r[...]
        h = jnp.dot(graph, wd1_r[...], preferred_element_type=F32) + bd1_r[...]
        out_r[...] = jnp.dot(h, wd2_r[...],
                             preferred_element_type=F32) + bd2_r[...]
        btt_r[...] = ((bz0_r[...] + be0_r[...]) + bz1_r[...]) + be1_r[...]
    return pl.pallas_call(body, out_shape=[
        jax.ShapeDtypeStruct((B, 2), F32),
        jax.ShapeDtypeStruct((1, 1), F32)])(
        zo, eo, nb, eb, wfus, bfus, wd1, bd1, wd2, bd2,
        btz0, bte0, btz1, bte1)


def kernel(x, hyperedge_feature, node_coord, W_n2e, W_e2n, w_coord, W_x, b_x,
           W_e, b_e, ln_g, ln_b, W_fus, b_fus, W_dec1, b_dec1, W_dec2, b_dec2,
           hyperedge_node_idx, hyperedge_edge_idx, node_batch_idx,
           hyperedge_batch_idx, XX_index, EE_index):
    hf = hyperedge_feature
    n_src, e_dst2 = _pad_pair(hyperedge_node_idx, hyperedge_edge_idx,
                              E_PAD, N, M)
    e_src, n_dst2 = _pad_pair(hyperedge_edge_idx, hyperedge_node_idx,
                              E_PAD, M, N)
    xx_src, xx_dst2 = _pad_pair(XX_index[0], XX_index[1], E_PAD, N, N)
    ee_src, ee_dst2 = _pad_pair(EE_index[0], EE_index[1], EE_PAD, M, M)

    ce_f, cn_f, dx_f, de_f, a1p = _segsum([
        (None, None, e_dst2, SD_M, 8),
        (None, None, n_dst2, SD_N, 8),
        (None, None, xx_dst2, SD_N, 8),
        (None, None, ee_dst2, SD_M, 8),
        (x, n_src, e_dst2, SD_M, 8),
    ])
    cep = ce_f[:, :M, 0:1]
    dxp = dx_f[:, :N, 0:1]
    dep = de_f[:, :M, 0:1]
    cnv = cn_f[:, :SD_N, 0:1]
    dxv = dx_f[:, :SD_N, 0:1]
    dev = de_f[:, :SD_M, 0:1]

    xs0, hs0, eagg = _t1p(a1p, cep, dxp, dep, x, hf, W_n2e[0])

    a3p, a4p, a2p = _segsum([
        (xs0, xx_src, xx_dst2, SD_N, 8),
        (hs0, ee_src, ee_dst2, SD_M, 8),
        (eagg, e_src, n_dst2, SD_N, 8),
    ])
    a2c = _comb(a2p, cnv, N, "mean")
    a3c = _comb(a3p, dxv, N, "rsqrt")
    a4c = _comb(a4p, dev, M, "rsqrt")
    zo0, xs1, btz0 = _t2z(a2c, a3c, x, dxp, W_e2n[0], W_x[0], b_x[0],
                          ln_g[0], ln_b[0], True)
    eo0, hs1, bte0 = _t2e(a4c, eagg, hf, dep, W_e[0], b_e[0],
                          ln_g[0], ln_b[0], True)

    a1p1, a3p1, a4p1 = _segsum([
        (zo0, n_src, e_dst2, SD_M, 8),
        (xs1, xx_src, xx_dst2, SD_N, 8),
        (hs1, ee_src, ee_dst2, SD_M, 8),
    ])
    eagg1 = _t1(a1p1, cep, W_n2e[1])

    (a2p1,) = _segsum([(eagg1, e_src, n_dst2, SD_N, 8)])
    a2c1 = _comb(a2p1, cnv, N, "mean")
    a3c1 = _comb(a3p1, dxv, N, "rsqrt")
    a4c1 = _comb(a4p1, dev, M, "rsqrt")
    zo1, btz1 = _t2z(a2c1, a3c1, zo0, dxp, W_e2n[1], W_x[1], b_x[1],
                     ln_g[1], ln_b[1], False)
    eo1, bte1 = _t2e(a4c1, eagg1, eo0, dep, W_e[1], b_e[1],
                     ln_g[1], ln_b[1], False)

    out, btt = _t3(zo1, eo1, node_batch_idx.astype(jnp.int32),
                   hyperedge_batch_idx.astype(jnp.int32),
                   W_fus[1], b_fus[1], W_dec1, b_dec1, W_dec2, b_dec2,
                   btz0, bte0, btz1, bte1)
    return out, btt.reshape(())


# speedup vs baseline: 1.1256x; 1.0639x over previous
import numpy as np
import jax
import jax.numpy as jnp
from jax import lax
from jax.experimental import pallas as pl
from jax.experimental.pallas import tpu as pltpu
from jax.experimental.pallas import tpu_sc as plsc

N = 10000
M = 2500
D = 128
B = 128
E = 320000
EEE = 80000

NC = 2
NS = 16
NW = NC * NS
CHUNK = 128

E_PAD = 327680
EE_PAD = 98304

SD_N = 10112
SD_M = 2560
SD16_N = 10240
SD16_M = 3072
OR_N = SD16_N * 16 // 128
OR_M = SD16_M * 16 // 128

F32 = jnp.float32


def _mesh():
    return plsc.VectorSubcoreMesh(core_axis_name="c", subcore_axis_name="s",
                                  num_cores=NC, num_subcores=NS)


def _pad_pair(src, dst, epad, table_rows, seg_rows):
    p = epad - src.shape[0]
    ar = jnp.arange(p, dtype=jnp.int32)
    src_p = jnp.concatenate([src.astype(jnp.int32), ar % min(2048, table_rows)])
    dst_p = jnp.concatenate([dst.astype(jnp.int32), seg_rows + (ar % 8)])
    return src_p, dst_p.reshape(epad // CHUNK, CHUNK)


def _make_segsum(ops):
    n_ops = len(ops)
    n_tab = sum(1 for o in ops if o[0])
    tab_pos = {}
    for i, o in enumerate(ops):
        if o[0]:
            tab_pos[i] = len(tab_pos)

    def body(*refs):
        tables = refs[0:n_tab]
        srcs = refs[n_tab:2 * n_tab]
        dsts = refs[2 * n_tab:2 * n_tab + n_ops]
        zeros = refs[2 * n_tab + n_ops]
        outs = refs[2 * n_tab + n_ops + 1: 2 * n_tab + 2 * n_ops + 1]
        it = iter(refs[2 * n_tab + 2 * n_ops + 1:])
        accum = next(it)
        sidx, didx, rows0, rows1, sem0, sem1 = (next(it), next(it), next(it),
                                                next(it), next(it), next(it))
        c = lax.axis_index("c")
        s = lax.axis_index("s")
        wid = s * NC + c

        if n_tab < n_ops:
            ones16 = jnp.ones((16,), F32)

            def fill(r):
                for jj in range(8):
                    rows0[r, pl.ds(16 * jj, 16)] = ones16
            pl.loop(0, CHUNK)(fill)

        for k in range(n_ops):
            has_tab, sd, epad, grp = ops[k]
            cw = epad // NW // CHUNK
            ng = cw // grp
            base = wid * cw

            if has_tab:
                rpt = sd // NS
                pltpu.sync_copy(zeros.at[pl.ds(0, rpt)],
                                accum.at[pl.ds(s * rpt, rpt)])
                plsc.subcore_barrier()
                t = tab_pos[k]

                def group_body(g, t=t, k=k, base=base, grp=grp):
                    gb = base + g * grp
                    pltpu.sync_copy(srcs[t].at[pl.ds(gb * CHUNK, grp * CHUNK)],
                                    sidx.at[pl.ds(0, grp * CHUNK)])
                    pltpu.sync_copy(dsts[k].at[pl.ds(gb, grp)],
                                    didx.at[pl.ds(0, grp)])
                    rbufs = (rows0, rows1)
                    gsems = (sem0, sem1)
                    gd = [None] * grp
                    gd[0] = pltpu.async_copy(
                        tables[t].at[sidx.at[pl.ds(0, CHUNK)]], rbufs[0],
                        gsems[0])
                    for j in range(grp):
                        if j + 1 < grp:
                            gd[j + 1] = pltpu.async_copy(
                                tables[t].at[
                                    sidx.at[pl.ds((j + 1) * CHUNK, CHUNK)]],
                                rbufs[(j + 1) % 2], gsems[(j + 1) % 2])
                        gd[j].wait()
                        pltpu.sync_copy(rbufs[j % 2], accum.at[didx.at[j]],
                                        add=True)

                pl.loop(0, ng)(group_body)
                plsc.subcore_barrier()

                pltpu.sync_copy(accum.at[pl.ds(s * rpt, rpt)],
                                outs[k].at[c].at[pl.ds(s * rpt, rpt)])
                plsc.subcore_barrier()
            else:
                rpt = sd // NS
                pltpu.sync_copy(zeros.at[pl.ds(0, rpt)],
                                accum.at[pl.ds(s * rpt, rpt)])
                plsc.subcore_barrier()

                def group_body(g, k=k, base=base, grp=grp):
                    gb = base + g * grp
                    pltpu.sync_copy(dsts[k].at[pl.ds(gb, grp)],
                                    didx.at[pl.ds(0, grp)])
                    for j in range(grp):
                        pltpu.sync_copy(rows0, accum.at[didx.at[j]],
                                        add=True)

                pl.loop(0, ng)(group_body)
                plsc.subcore_barrier()

                pltpu.sync_copy(accum.at[pl.ds(s * rpt, rpt)],
                                outs[k].at[c].at[pl.ds(s * rpt, rpt)])
                plsc.subcore_barrier()

    max_grp = max(o[3] for o in ops)
    max_sd = max(o[1] for o in ops)
    scratch = [pltpu.VMEM_SHARED((max_sd, D), F32),
               pltpu.VMEM((max_grp * CHUNK,), jnp.int32),
               pltpu.VMEM((max_grp, CHUNK), jnp.int32),
               pltpu.VMEM((CHUNK, D), F32),
               pltpu.VMEM((CHUNK, D), F32),
               pltpu.SemaphoreType.DMA,
               pltpu.SemaphoreType.DMA]

    out_type = [jax.ShapeDtypeStruct((NC, o[1], D), F32) for o in ops]
    return pl.kernel(body, out_type=out_type, mesh=_mesh(),
                     scratch_types=scratch)


def _segsum(op_args):
    ops = [(a[0] is not None, a[3], a[2].shape[0] * CHUNK, a[4])
           for a in op_args]
    fn = _make_segsum(ops)
    zeros = jnp.zeros((max(o[1] for o in ops) // NS, D), F32)
    args = ([a[0] for a in op_args if a[0] is not None]
            + [a[1] for a in op_args if a[1] is not None]
            + [a[2] for a in op_args] + [zeros])
    res = fn(*args)
    return res if isinstance(res, (list, tuple)) else (res,)


SQRT2 = float(np.sqrt(2.0))


def _gelu(z):
    return 0.5 * z * (1.0 + lax.erf(z / SQRT2))


def _layer_norm(z, g, b):
    mu = jnp.mean(z, axis=-1, keepdims=True)
    v = jnp.mean((z - mu) ** 2, axis=-1, keepdims=True)
    return (z - mu) / jnp.sqrt(v + 1e-5) * g + b


def _bt_loss(z1, z2, n):
    m1 = jnp.mean(z1, axis=0, keepdims=True)
    s1 = jnp.sqrt(jnp.sum((z1 - m1) ** 2, axis=0, keepdims=True) / (n - 1))
    z1 = (z1 - m1) / s1
    m2 = jnp.mean(z2, axis=0, keepdims=True)
    s2 = jnp.sqrt(jnp.sum((z2 - m2) ** 2, axis=0, keepdims=True) / (n - 1))
    z2 = (z2 - m2) / s2

    def bn(z):
        mu = jnp.mean(z, axis=0, keepdims=True)
        v = jnp.mean((z - mu) ** 2, axis=0, keepdims=True)
        return (z - mu) / jnp.sqrt(v + 1e-5)

    cc = lax.dot_general(bn(z1), bn(z2), (((0,), (0,)), ((), ())),
                         preferred_element_type=F32) / float(B)
    eye = (lax.broadcasted_iota(jnp.int32, (D, D), 0)
           == lax.broadcasted_iota(jnp.int32, (D, D), 1)).astype(F32)
    on = jnp.sum(((cc - 1.0) * eye) ** 2)
    csq = cc * cc
    off = jnp.sum(csq) - jnp.sum(csq * eye)
    return on + 0.005 * off


def _cnt(ref):
    return jnp.maximum(ref[0, :, 0:1] + ref[1, :, 0:1], 1.0)


def _t1p(a1p, cep, dxp, dep, x, hf, w):
    def body(a_r, c_r, dx_r, de_r, x_r, hf_r, w_r, xs_r, hs_r, eagg_r):
        xs_r[...] = x_r[...] / jnp.sqrt(_cnt(dx_r))
        hs_r[...] = hf_r[...] / jnp.sqrt(_cnt(de_r))
        sums = a_r[0, :M, :] + a_r[1, :M, :]
        eagg_r[...] = jnp.dot(sums / _cnt(c_r), w_r[...],
                              preferred_element_type=F32)
    return pl.pallas_call(body, out_shape=[
        jax.ShapeDtypeStruct((N, D), F32),
        jax.ShapeDtypeStruct((M, D), F32),
        jax.ShapeDtypeStruct((M, D), F32)])(a1p, cep, dxp, dep, x, hf, w)


def _t1(a1p, cep, w):
    def body(a_r, c_r, w_r, o_r):
        sums = a_r[0, :M, :] + a_r[1, :M, :]
        o_r[...] = jnp.dot(sums / _cnt(c_r), w_r[...],
                           preferred_element_type=F32)
    return pl.pallas_call(body, out_shape=jax.ShapeDtypeStruct((M, D), F32))(
        a1p, cep, w)


def _comb(p, v, s_rows, mode):
    sd = p.shape[1]
    rb = 1264 if sd == SD_N else 1280
    def body(p_r, v_r, o_r):
        cnt = jnp.maximum(v_r[0] + v_r[1], 1.0)
        sums = p_r[0] + p_r[1]
        o_r[...] = sums / cnt if mode == "mean" else sums / jnp.sqrt(cnt)
    out = pl.pallas_call(
        body,
        grid=(sd // rb,),
        in_specs=[pl.BlockSpec((NC, rb, D), lambda i: (0, i, 0)),
                  pl.BlockSpec((NC, rb, 1), lambda i: (0, i, 0))],
        out_specs=pl.BlockSpec((rb, D), lambda i: (i, 0)),
        out_shape=jax.ShapeDtypeStruct((sd, D), F32))(p, v)
    return out[:s_rows]


def _t2z(a2c, a3c, x_res, dxp, we2n, wx, bx, lng, lnb, want_next):
    def body(a2_r, a3_r, x_r, dx_r, we2n_r, wx_r, bx_r, lng_r, lnb_r, *outs):
        z = jnp.dot(a2_r[...], we2n_r[...], preferred_element_type=F32)
        z_imp = jnp.dot(a3_r[...], wx_r[...],
                        preferred_element_type=F32) + bx_r[...]
        bt = _bt_loss(z_imp, z, N)
        zo = _gelu(_layer_norm(z, lng_r[...], lnb_r[...])) + x_r[...]
        outs[0][...] = zo
        outs[-1][...] = bt.reshape(1, 1)
        if want_next:
            outs[1][...] = zo / jnp.sqrt(_cnt(dx_r))
    shapes = [jax.ShapeDtypeStruct((N, D), F32)]
    if want_next:
        shapes.append(jax.ShapeDtypeStruct((N, D), F32))
    shapes.append(jax.ShapeDtypeStruct((1, 1), F32))
    return pl.pallas_call(body, out_shape=shapes)(
        a2c, a3c, x_res, dxp, we2n, wx, bx, lng, lnb)


def _t2e(a4c, eagg, hf_res, dep, we, be, lng, lnb, want_next):
    def body(a4_r, eagg_r, hf_r, de_r, we_r, be_r, lng_r, lnb_r, *outs):
        e_imp = jnp.dot(a4_r[...], we_r[...],
                        preferred_element_type=F32) + be_r[...]
        e = eagg_r[...]
        bt = _bt_loss(e_imp, e, M)
        eo = _gelu(_layer_norm(e, lng_r[...], lnb_r[...])) + hf_r[...]
        outs[0][...] = eo
        outs[-1][...] = bt.reshape(1, 1)
        if want_next:
            outs[1][...] = eo / jnp.sqrt(_cnt(de_r))
    shapes = [jax.ShapeDtypeStruct((M, D), F32)]
    if want_next:
        shapes.append(jax.ShapeDtypeStruct((M, D), F32))
    shapes.append(jax.ShapeDtypeStruct((1, 1), F32))
    return pl.pallas_call(body, out_shape=shapes)(
        a4c, eagg, hf_res, dep, we, be, lng, lnb)


def _t3(zo, eo, nb, eb, wfus, bfus, wd1, bd1, wd2, bd2,
        btz0, bte0, btz1, bte1):
    def body(zo_r, eo_r, nb_r, eb_r, wfus_r, bfus_r, wd1_r, bd1_r, wd2_r,
             bd2_r, bz0_r, be0_r, bz1_r, be1_r, out_r, btt_r):
        def pool(idx, data):
            oh = (idx.reshape(idx.shape[0], 1)
                  == lax.broadcasted_iota(jnp.int32, (1, B), 1)).astype(F32)
            sums = lax.dot_general(oh, data, (((0,), (0,)), ((), ())),
                                   preferred_element_type=F32)
            cnt = jnp.maximum(jnp.sum(oh, axis=0), 1.0).reshape(B, 1)
            return sums / cnt

        zg = pool(nb_r[...], zo_r[...])
        eg = pool(eb_r[...], eo_r[...])
        graph = jnp.dot(jnp.concatenate([zg, eg], axis=1), wfus_r[...],
                        preferred_element_type=F32) + bfus_r[...]
        h = jnp.dot(graph, wd1_r[...], preferred_element_type=F32) + bd1_r[...]
        out_r[...] = jnp.dot(h, wd2_r[...],
                             preferred_element_type=F32) + bd2_r[...]
        btt_r[...] = ((bz0_r[...] + be0_r[...]) + bz1_r[...]) + be1_r[...]
    return pl.pallas_call(body, out_shape=[
        jax.ShapeDtypeStruct((B, 2), F32),
        jax.ShapeDtypeStruct((1, 1), F32)])(
        zo, eo, nb, eb, wfus, bfus, wd1, bd1, wd2, bd2,
        btz0, bte0, btz1, bte1)


def kernel(x, hyperedge_feature, node_coord, W_n2e, W_e2n, w_coord, W_x, b_x,
           W_e, b_e, ln_g, ln_b, W_fus, b_fus, W_dec1, b_dec1, W_dec2, b_dec2,
           hyperedge_node_idx, hyperedge_edge_idx, node_batch_idx,
           hyperedge_batch_idx, XX_index, EE_index):
    hf = hyperedge_feature
    n_src, e_dst2 = _pad_pair(hyperedge_node_idx, hyperedge_edge_idx,
                              E_PAD, N, M)
    e_src, n_dst2 = _pad_pair(hyperedge_edge_idx, hyperedge_node_idx,
                              E_PAD, M, N)
    xx_src, xx_dst2 = _pad_pair(XX_index[0], XX_index[1], E_PAD, N, N)
    ee_src, ee_dst2 = _pad_pair(EE_index[0], EE_index[1], EE_PAD, M, M)

    ce_f, cn_f, dx_f, de_f, a1p = _segsum([
        (None, None, e_dst2, SD_M, 8),
        (None, None, n_dst2, SD_N, 8),
        (None, None, xx_dst2, SD_N, 8),
        (None, None, ee_dst2, SD_M, 8),
        (x, n_src, e_dst2, SD_M, 8),
    ])
    cep = ce_f[:, :M, 0:1]
    dxp = dx_f[:, :N, 0:1]
    dep = de_f[:, :M, 0:1]
    cnv = cn_f[:, :SD_N, 0:1]
    dxv = dx_f[:, :SD_N, 0:1]
    dev = de_f[:, :SD_M, 0:1]

    xs0, hs0, eagg = _t1p(a1p, cep, dxp, dep, x, hf, W_n2e[0])

    a3p, a4p, a2p = _segsum([
        (xs0, xx_src, xx_dst2, SD_N, 8),
        (hs0, ee_src, ee_dst2, SD_M, 8),
        (eagg, e_src, n_dst2, SD_N, 8),
    ])
    a2c = _comb(a2p, cnv, N, "mean")
    a3c = _comb(a3p, dxv, N, "rsqrt")
    a4c = _comb(a4p, dev, M, "rsqrt")
    zo0, xs1, btz0 = _t2z(a2c, a3c, x, dxp, W_e2n[0], W_x[0], b_x[0],
                          ln_g[0], ln_b[0], True)
    eo0, hs1, bte0 = _t2e(a4c, eagg, hf, dep, W_e[0], b_e[0],
                          ln_g[0], ln_b[0], True)

    a1p1, a3p1, a4p1 = _segsum([
        (zo0, n_src, e_dst2, SD_M, 8),
        (xs1, xx_src, xx_dst2, SD_N, 8),
        (hs1, ee_src, ee_dst2, SD_M, 8),
    ])
    eagg1 = _t1(a1p1, cep, W_n2e[1])

    (a2p1,) = _segsum([(eagg1, e_src, n_dst2, SD_N, 8)])
    a2c1 = _comb(a2p1, cnv, N, "mean")
    a3c1 = _comb(a3p1, dxv, N, "rsqrt")
    a4c1 = _comb(a4p1, dev, M, "rsqrt")
    zo1, btz1 = _t2z(a2c1, a3c1, zo0, dxp, W_e2n[1], W_x[1], b_x[1],
                     ln_g[1], ln_b[1], False)
    eo1, bte1 = _t2e(a4c1, eagg1, eo0, dep, W_e[1], b_e[1],
                     ln_g[1], ln_b[1], False)

    out, btt = _t3(zo1, eo1, node_batch_idx.astype(jnp.int32),
                   hyperedge_batch_idx.astype(jnp.int32),
                   W_fus[1], b_fus[1], W_dec1, b_dec1, W_dec2, b_dec2,
                   btz0, bte0, btz1, bte1)
    return out, btt.reshape(())
